# Initial kernel scaffold; baseline (speedup 1.0000x reference)
import jax
import jax.numpy as jnp
from jax.experimental import pallas as pl


def kernel(P3, P4, P5, budget, W1_P3, b1_P3, W2_P3, b2_P3, W1_P4, b1_P4, W2_P4, b2_P4, W1_P5, b1_P5, W2_P5, b2_P5):
    raise NotImplementedError("write your pallas kernel here")


# speedup vs baseline: 1.2221x; 1.2221x over previous
import functools

import jax
import jax.numpy as jnp
from jax import lax
from jax.experimental import pallas as pl

_K = 16


def _level_body(x_ref, w1_ref, b1_ref, w2_ref, y_ref, m_ref, *, n, bb):
    xb = x_ref[0]
    w1 = w1_ref[...]
    h = lax.dot_general(w1, xb, dimension_numbers=(((1,), (1,)), ((), ())),
                        preferred_element_type=jnp.float32)
    h = h + b1_ref[0][:, None, None]
    h = 0.5 * h * (1.0 + lax.erf(h * 0.7071067811865476))
    w2 = w2_ref[...]
    s = lax.dot_general(w2, h, dimension_numbers=(((1,), (0,)), ((), ())),
                        preferred_element_type=jnp.float32)
    scores = s[0]

    iota = lax.broadcasted_iota(jnp.int32, (bb, n), 1)

    def step(_, taken):
        cur = jnp.where(taken, -jnp.inf, scores)
        m = jnp.max(cur, axis=1, keepdims=True)
        idx = jnp.min(jnp.where(cur == m, iota, n), axis=1, keepdims=True)
        return taken | (iota == idx)

    taken = lax.fori_loop(0, _K, step, jnp.zeros((bb, n), jnp.bool_),
                          unroll=True)
    mf = taken.astype(jnp.float32)
    m_ref[0] = mf
    y_ref[0] = xb * mf[:, None, :]


def _run_level(x, w1, b1, w2, groups):
    b, c, hh, ww = x.shape
    n = hh * ww
    bb = b // groups
    x4 = x.reshape(groups, bb, c, n)
    body = functools.partial(_level_body, n=n, bb=bb)
    y4, m3 = pl.pallas_call(
        body,
        grid=(groups,),
        in_specs=[
            pl.BlockSpec((1, bb, c, n), lambda g: (g, 0, 0, 0)),
            pl.BlockSpec((64, 128), lambda g: (0, 0)),
            pl.BlockSpec((1, 64), lambda g: (0, 0)),
            pl.BlockSpec((1, 64), lambda g: (0, 0)),
        ],
        out_specs=[
            pl.BlockSpec((1, bb, c, n), lambda g: (g, 0, 0, 0)),
            pl.BlockSpec((1, bb, n), lambda g: (g, 0, 0)),
        ],
        out_shape=[
            jax.ShapeDtypeStruct((groups, bb, c, n), jnp.float32),
            jax.ShapeDtypeStruct((groups, bb, n), jnp.float32),
        ],
    )(x4, w1, b1.reshape(1, 64), w2)
    y = y4.reshape(b, c, hh, ww)
    mask = m3.reshape(b, n).astype(jnp.bool_)
    return y, mask


def kernel(P3, P4, P5, budget, W1_P3, b1_P3, W2_P3, b2_P3,
           W1_P4, b1_P4, W2_P4, b2_P4, W1_P5, b1_P5, W2_P5, b2_P5):
    y3, m3 = _run_level(P3, W1_P3, b1_P3, W2_P3, groups=4)
    y4, m4 = _run_level(P4, W1_P4, b1_P4, W2_P4, groups=2)
    y5, m5 = _run_level(P5, W1_P5, b1_P5, W2_P5, groups=1)
    k = jnp.array([_K], dtype=jnp.int32)
    return (y3, y4, y5, m3, m4, m5, k, k, k)
